# baseline (speedup 1.0000x reference)
import functools

import jax
import jax.numpy as jnp
from jax import lax
from jax.experimental import pallas as pl
from jax.experimental.pallas import tpu as pltpu
from jax.experimental.pallas import tpu_sc as plsc


_ROWS = 400


def _mm2_body(y_ref, wl_ref, wr_ref, xl_ref, xr_ref):
    yb = y_ref[...]
    xl_ref[...] = jnp.dot(yb, wl_ref[...], preferred_element_type=jnp.float32)
    xr_ref[...] = jnp.dot(yb, wr_ref[...], preferred_element_type=jnp.float32)


def _mm2(y, Wl, Wr):
    n, p = y.shape
    d = Wl.shape[1]
    grid = n // _ROWS
    return pl.pallas_call(
        _mm2_body,
        grid=(grid,),
        in_specs=[
            pl.BlockSpec((_ROWS, p), lambda i: (i, 0)),
            pl.BlockSpec((p, d), lambda i: (0, 0)),
            pl.BlockSpec((p, d), lambda i: (0, 0)),
        ],
        out_specs=[
            pl.BlockSpec((_ROWS, d), lambda i: (i, 0)),
            pl.BlockSpec((_ROWS, d), lambda i: (i, 0)),
        ],
        out_shape=[
            jax.ShapeDtypeStruct((n, d), jnp.float32),
            jax.ShapeDtypeStruct((n, d), jnp.float32),
        ],
    )(y, Wl, Wr)


def _mid_body(a0_ref, a1_ref, exp_ref, b_ref, wl_ref, wr_ref, xl_ref, xr_ref):
    s = a0_ref[...] + a1_ref[...]
    msg = s[:, :128]
    r8 = 1.0 / (s[:, 128:136] + 1e-16)
    rw = jnp.dot(r8, exp_ref[...], preferred_element_type=jnp.float32)
    out1 = msg * rw + b_ref[...][0:1, :]
    h = jnp.where(out1 > 0.0, out1, jnp.exp(out1) - 1.0)
    xl_ref[...] = jnp.dot(h, wl_ref[...], preferred_element_type=jnp.float32)
    xr_ref[...] = jnp.dot(h, wr_ref[...], preferred_element_type=jnp.float32)


def _mid(a0, a1, expand, b1, Wl2, Wr2):
    n, w = a0.shape
    z = Wl2.shape[1]
    grid = n // _ROWS
    return pl.pallas_call(
        _mid_body,
        grid=(grid,),
        in_specs=[
            pl.BlockSpec((_ROWS, w), lambda i: (i, 0)),
            pl.BlockSpec((_ROWS, w), lambda i: (i, 0)),
            pl.BlockSpec((8, 128), lambda i: (0, 0)),
            pl.BlockSpec((8, 128), lambda i: (0, 0)),
            pl.BlockSpec((128, z), lambda i: (0, 0)),
            pl.BlockSpec((128, z), lambda i: (0, 0)),
        ],
        out_specs=[
            pl.BlockSpec((_ROWS, z), lambda i: (i, 0)),
            pl.BlockSpec((_ROWS, z), lambda i: (i, 0)),
        ],
        out_shape=[
            jax.ShapeDtypeStruct((n, z), jnp.float32),
            jax.ShapeDtypeStruct((n, z), jnp.float32),
        ],
    )(a0, a1, expand, b1, Wl2, Wr2)


def _fin_body(a0_ref, a1_ref, b_ref, z_ref):
    s = a0_ref[...] + a1_ref[...]
    msg = s[:, :64]
    r = 1.0 / (s[:, 64:65] + 1e-16)
    z_ref[...] = msg * r + b_ref[...][0:1, :]


def _fin(a0, a1, b2):
    n, w = a0.shape
    z = b2.shape[1]
    grid = n // _ROWS
    return pl.pallas_call(
        _fin_body,
        grid=(grid,),
        in_specs=[
            pl.BlockSpec((_ROWS, w), lambda i: (i, 0)),
            pl.BlockSpec((_ROWS, w), lambda i: (i, 0)),
            pl.BlockSpec((8, z), lambda i: (0, 0)),
        ],
        out_specs=pl.BlockSpec((_ROWS, z), lambda i: (i, 0)),
        out_shape=jax.ShapeDtypeStruct((n, z), jnp.float32),
    )(a0, a1, b2)


_NC, _NS = 2, 16
_K = 40


def _vperm(v, idx):
    return lax.gather(
        v, idx[:, None],
        lax.GatherDimensionNumbers(
            offset_dims=(), collapsed_slice_dims=(0,), start_index_map=(0,)),
        slice_sizes=(1,),
        mode=lax.GatherScatterMode.PROMISE_IN_BOUNDS)


def _make_edge_kernel(N, E, D, H):
    ACCW = D + 16
    NW = _NC * _NS
    EPT = E // NW
    NCH = EPT // _K
    NPT = N // _NS
    NJ = D // 16
    JH = NJ // H
    assert EPT % _K == 0 and NCH % 2 == 0 and NJ % H == 0

    mesh = plsc.VectorSubcoreMesh(core_axis_name="c", subcore_axis_name="s")

    @functools.partial(
        pl.kernel,
        out_type=jax.ShapeDtypeStruct((_NC, N, ACCW), jnp.float32),
        mesh=mesh,
        compiler_params=pltpu.CompilerParams(
            use_tc_tiling_on_sc=False, needs_layout_passes=False),
        scratch_types=[
            pltpu.VMEM_SHARED((N, ACCW), jnp.float32),
            pltpu.VMEM((_K,), jnp.int32),
            pltpu.VMEM((_K,), jnp.int32),
            pltpu.VMEM((_K,), jnp.int32),
            pltpu.VMEM((_K,), jnp.int32),
            pltpu.VMEM((_K, D), jnp.float32),
            pltpu.VMEM((_K, D), jnp.float32),
            pltpu.VMEM((_K, D), jnp.float32),
            pltpu.VMEM((_K, D), jnp.float32),
            pltpu.VMEM((_K, ACCW), jnp.float32),
            pltpu.VMEM((D,), jnp.float32),
            pltpu.SemaphoreType.DMA,
            pltpu.SemaphoreType.DMA,
            pltpu.SemaphoreType.DMA,
            pltpu.SemaphoreType.DMA,
        ],
    )
    def ek(xl_hbm, xr_hbm, ei_hbm, att_hbm, out_hbm,
           acc, src_a, dst_a, src_b, dst_b, xlb_a, xrb_a, xlb_b, xrb_b,
           msgb, attb, sxl_a, sxr_a, sxl_b, sxr_b):
        cid = lax.axis_index("c")
        sid = lax.axis_index("s")
        wid = cid * _NS + sid
        zv = jnp.zeros((16,), jnp.float32)
        lane = lax.iota(jnp.int32, 16)

        nseg = ACCW // 16

        def zbody(t, carry):
            msgb[t // nseg, pl.ds((t % nseg) * 16, 16)] = zv
            return carry

        lax.fori_loop(0, _K * nseg, zbody, 0)
        for j in range(NPT // _K):
            pltpu.sync_copy(msgb, acc.at[pl.ds(sid * NPT + j * _K, _K)])
        zrem = NPT % _K
        if zrem:
            pltpu.sync_copy(
                msgb.at[pl.ds(0, zrem)],
                acc.at[pl.ds(sid * NPT + (NPT // _K) * _K, zrem)])
        pltpu.sync_copy(att_hbm, attb)
        plsc.subcore_barrier()

        attv = [attb[pl.ds(16 * j, 16)] for j in range(NJ)]
        perms = [lane ^ 8, lane ^ 4, lane ^ 2, lane ^ 1]

        bufa = (src_a, dst_a, xlb_a, xrb_a, sxl_a, sxr_a)
        bufb = (src_b, dst_b, xlb_b, xrb_b, sxl_b, sxr_b)

        def issue(i, buf):
            src_v, dst_v, xlb, xrb, sxl, sxr = buf
            ebase = wid * EPT + i * _K
            pltpu.sync_copy(ei_hbm.at[0, pl.ds(ebase, _K)], src_v)
            pltpu.sync_copy(ei_hbm.at[1, pl.ds(ebase, _K)], dst_v)
            pltpu.async_copy(xl_hbm.at[src_v], xlb, sxl)
            pltpu.async_copy(xr_hbm.at[dst_v], xrb, sxr)

        def step(i, cur, nxt):
            src_v, dst_v, xlb, xrb, sxl, sxr = cur

            @pl.when(i + 1 < NCH)
            def _():
                issue(i + 1, nxt)

            pltpu.make_async_copy(xl_hbm.at[src_v], xlb, sxl).wait()
            pltpu.make_async_copy(xr_hbm.at[dst_v], xrb, sxr).wait()

            def edge(k):
                xs = [xlb[k, pl.ds(16 * j, 16)] for j in range(NJ)]
                ts = []
                for j in range(NJ):
                    s = xs[j] + xrb[k, pl.ds(16 * j, 16)]
                    m = jnp.maximum(s, 0.2 * s)
                    ts.append(m * attv[j])
                exb = []
                for h in range(H):
                    u = ts[h * JH]
                    for q in range(1, JH):
                        u = u + ts[h * JH + q]
                    for p in perms:
                        u = u + _vperm(u, p)
                    exb.append(jnp.exp(u))
                for j in range(NJ):
                    msgb[k, pl.ds(16 * j, 16)] = xs[j] * exb[j // JH]
                evec = zv
                for h in range(H):
                    evec = jnp.where(lane == h, exb[h], evec)
                msgb[k, pl.ds(D, 16)] = evec

            plsc.parallel_loop(0, _K, 1, unroll=4)(lambda k: edge(k))
            pltpu.sync_copy(msgb, acc.at[dst_v], add=True)

        issue(0, bufa)

        def chunk2(t, carry):
            step(2 * t, bufa, bufb)
            step(2 * t + 1, bufb, bufa)
            return carry

        lax.fori_loop(0, NCH // 2, chunk2, 0)
        plsc.subcore_barrier()
        pltpu.sync_copy(acc.at[pl.ds(sid * NPT, NPT)],
                        out_hbm.at[cid, pl.ds(sid * NPT, NPT)])

    return ek


def kernel(y, y_edge_index, Wl1, Wr1, att1, b1, Wl2, Wr2, att2, b2):
    n = y.shape[0]
    e = y_edge_index.shape[1]
    ei = y_edge_index.astype(jnp.int32)

    expand = jnp.zeros((8, 128), jnp.float32)
    hidx = jnp.arange(128) // 32
    expand = expand.at[hidx, jnp.arange(128)].set(1.0)

    b1m = jnp.broadcast_to(b1.reshape(1, -1), (8, 128))
    b2m = jnp.broadcast_to(b2.reshape(1, -1), (8, 64))

    xl1, xr1 = _mm2(y, Wl1, Wr1)
    acc1 = _make_edge_kernel(n, e, 128, 4)(xl1, xr1, ei, att1.reshape(-1))
    xl2, xr2 = _mid(acc1[0], acc1[1], expand, b1m, Wl2, Wr2)

    acc2 = _make_edge_kernel(n, e, 64, 1)(xl2, xr2, ei, att2.reshape(-1))
    return _fin(acc2[0], acc2[1], b2m)


# speedup vs baseline: 1.0244x; 1.0244x over previous
import functools

import jax
import jax.numpy as jnp
from jax import lax
from jax.experimental import pallas as pl
from jax.experimental.pallas import tpu as pltpu
from jax.experimental.pallas import tpu_sc as plsc


_ROWS = 400


def _mm2_body(y_ref, wl_ref, wr_ref, xl_ref, xr_ref):
    yb = y_ref[...]
    xl_ref[...] = jnp.dot(yb, wl_ref[...], preferred_element_type=jnp.float32)
    xr_ref[...] = jnp.dot(yb, wr_ref[...], preferred_element_type=jnp.float32)


def _mm2(y, Wl, Wr):
    n, p = y.shape
    d = Wl.shape[1]
    grid = n // _ROWS
    return pl.pallas_call(
        _mm2_body,
        grid=(grid,),
        in_specs=[
            pl.BlockSpec((_ROWS, p), lambda i: (i, 0)),
            pl.BlockSpec((p, d), lambda i: (0, 0)),
            pl.BlockSpec((p, d), lambda i: (0, 0)),
        ],
        out_specs=[
            pl.BlockSpec((_ROWS, d), lambda i: (i, 0)),
            pl.BlockSpec((_ROWS, d), lambda i: (i, 0)),
        ],
        out_shape=[
            jax.ShapeDtypeStruct((n, d), jnp.float32),
            jax.ShapeDtypeStruct((n, d), jnp.float32),
        ],
    )(y, Wl, Wr)


def _mid_body(a0_ref, a1_ref, exp_ref, b_ref, wl_ref, wr_ref, xl_ref, xr_ref):
    s = a0_ref[...] + a1_ref[...]
    msg = s[:, :128]
    r8 = 1.0 / (s[:, 128:136] + 1e-16)
    rw = jnp.dot(r8, exp_ref[...], preferred_element_type=jnp.float32)
    out1 = msg * rw + b_ref[...][0:1, :]
    h = jnp.where(out1 > 0.0, out1, jnp.exp(out1) - 1.0)
    xl_ref[...] = jnp.dot(h, wl_ref[...], preferred_element_type=jnp.float32)
    xr_ref[...] = jnp.dot(h, wr_ref[...], preferred_element_type=jnp.float32)


def _mid(a0, a1, expand, b1, Wl2, Wr2):
    n, w = a0.shape
    z = Wl2.shape[1]
    grid = n // _ROWS
    return pl.pallas_call(
        _mid_body,
        grid=(grid,),
        in_specs=[
            pl.BlockSpec((_ROWS, w), lambda i: (i, 0)),
            pl.BlockSpec((_ROWS, w), lambda i: (i, 0)),
            pl.BlockSpec((8, 128), lambda i: (0, 0)),
            pl.BlockSpec((8, 128), lambda i: (0, 0)),
            pl.BlockSpec((128, z), lambda i: (0, 0)),
            pl.BlockSpec((128, z), lambda i: (0, 0)),
        ],
        out_specs=[
            pl.BlockSpec((_ROWS, z), lambda i: (i, 0)),
            pl.BlockSpec((_ROWS, z), lambda i: (i, 0)),
        ],
        out_shape=[
            jax.ShapeDtypeStruct((n, z), jnp.float32),
            jax.ShapeDtypeStruct((n, z), jnp.float32),
        ],
    )(a0, a1, expand, b1, Wl2, Wr2)


def _fin_body(a0_ref, a1_ref, b_ref, z_ref):
    s = a0_ref[...] + a1_ref[...]
    msg = s[:, :64]
    r = 1.0 / (s[:, 64:65] + 1e-16)
    z_ref[...] = msg * r + b_ref[...][0:1, :]


def _fin(a0, a1, b2):
    n, w = a0.shape
    z = b2.shape[1]
    grid = n // _ROWS
    return pl.pallas_call(
        _fin_body,
        grid=(grid,),
        in_specs=[
            pl.BlockSpec((_ROWS, w), lambda i: (i, 0)),
            pl.BlockSpec((_ROWS, w), lambda i: (i, 0)),
            pl.BlockSpec((8, z), lambda i: (0, 0)),
        ],
        out_specs=pl.BlockSpec((_ROWS, z), lambda i: (i, 0)),
        out_shape=jax.ShapeDtypeStruct((n, z), jnp.float32),
    )(a0, a1, b2)


_NC, _NS = 2, 16
_K = 40


def _vperm(v, idx):
    return lax.gather(
        v, idx[:, None],
        lax.GatherDimensionNumbers(
            offset_dims=(), collapsed_slice_dims=(0,), start_index_map=(0,)),
        slice_sizes=(1,),
        mode=lax.GatherScatterMode.PROMISE_IN_BOUNDS)


def _make_edge_kernel(N, E, D, H):
    ACCW = D + 16
    NW = _NC * _NS
    EPT = E // NW
    NCH = EPT // _K
    NPT = N // _NS
    NJ = D // 16
    JH = NJ // H
    assert EPT % _K == 0 and NCH % 2 == 0 and NJ % H == 0

    mesh = plsc.VectorSubcoreMesh(core_axis_name="c", subcore_axis_name="s")

    @functools.partial(
        pl.kernel,
        out_type=jax.ShapeDtypeStruct((_NC, N, ACCW), jnp.float32),
        mesh=mesh,
        compiler_params=pltpu.CompilerParams(
            use_tc_tiling_on_sc=False, needs_layout_passes=False),
        scratch_types=[
            pltpu.VMEM_SHARED((N, ACCW), jnp.float32),
            pltpu.VMEM((_K,), jnp.int32),
            pltpu.VMEM((_K,), jnp.int32),
            pltpu.VMEM((_K,), jnp.int32),
            pltpu.VMEM((_K,), jnp.int32),
            pltpu.VMEM((_K, D), jnp.float32),
            pltpu.VMEM((_K, D), jnp.float32),
            pltpu.VMEM((_K, D), jnp.float32),
            pltpu.VMEM((_K, D), jnp.float32),
            pltpu.VMEM((_K, ACCW), jnp.float32),
            pltpu.VMEM((D,), jnp.float32),
            pltpu.SemaphoreType.DMA,
            pltpu.SemaphoreType.DMA,
            pltpu.SemaphoreType.DMA,
            pltpu.SemaphoreType.DMA,
        ],
    )
    def ek(xl_hbm, xr_hbm, ei_hbm, att_hbm, out_hbm,
           acc, src_a, dst_a, src_b, dst_b, xlb_a, xrb_a, xlb_b, xrb_b,
           msgb, attb, sxl_a, sxr_a, sxl_b, sxr_b):
        cid = lax.axis_index("c")
        sid = lax.axis_index("s")
        wid = cid * _NS + sid
        zv = jnp.zeros((16,), jnp.float32)
        lane = lax.iota(jnp.int32, 16)

        nseg = ACCW // 16

        def zbody(t, carry):
            msgb[t // nseg, pl.ds((t % nseg) * 16, 16)] = zv
            return carry

        lax.fori_loop(0, _K * nseg, zbody, 0)
        for j in range(NPT // _K):
            pltpu.sync_copy(msgb, acc.at[pl.ds(sid * NPT + j * _K, _K)])
        zrem = NPT % _K
        if zrem:
            pltpu.sync_copy(
                msgb.at[pl.ds(0, zrem)],
                acc.at[pl.ds(sid * NPT + (NPT // _K) * _K, zrem)])
        pltpu.sync_copy(att_hbm, attb)
        plsc.subcore_barrier()

        attv = [attb[pl.ds(16 * j, 16)] for j in range(NJ)]
        perms = [lane ^ 8, lane ^ 4, lane ^ 2, lane ^ 1]

        bufa = (src_a, dst_a, xlb_a, xrb_a, sxl_a, sxr_a)
        bufb = (src_b, dst_b, xlb_b, xrb_b, sxl_b, sxr_b)

        def issue(i, buf):
            src_v, dst_v, xlb, xrb, sxl, sxr = buf
            ebase = wid * EPT + i * _K
            pltpu.sync_copy(ei_hbm.at[0, pl.ds(ebase, _K)], src_v)
            pltpu.sync_copy(ei_hbm.at[1, pl.ds(ebase, _K)], dst_v)
            pltpu.async_copy(xl_hbm.at[src_v], xlb, sxl)
            pltpu.async_copy(xr_hbm.at[dst_v], xrb, sxr)

        def step(i, cur, nxt):
            src_v, dst_v, xlb, xrb, sxl, sxr = cur

            @pl.when(i + 1 < NCH)
            def _():
                issue(i + 1, nxt)

            pltpu.make_async_copy(xl_hbm.at[src_v], xlb, sxl).wait()
            pltpu.make_async_copy(xr_hbm.at[dst_v], xrb, sxr).wait()

            def edge(k):
                xs = [xlb[k, pl.ds(16 * j, 16)] for j in range(NJ)]
                ts = []
                for j in range(NJ):
                    s = xs[j] + xrb[k, pl.ds(16 * j, 16)]
                    m = jnp.maximum(s, 0.2 * s)
                    ts.append(m * attv[j])
                exb = []
                for h in range(H):
                    u = ts[h * JH]
                    for q in range(1, JH):
                        u = u + ts[h * JH + q]
                    eh = jnp.sum(u)
                    exb.append(jnp.exp(jnp.full((16,), eh, jnp.float32)))
                for j in range(NJ):
                    msgb[k, pl.ds(16 * j, 16)] = xs[j] * exb[j // JH]
                evec = zv
                for h in range(H):
                    evec = jnp.where(lane == h, exb[h], evec)
                msgb[k, pl.ds(D, 16)] = evec

            plsc.parallel_loop(0, _K, 1, unroll=8)(lambda k: edge(k))
            pltpu.sync_copy(msgb, acc.at[dst_v], add=True)

        issue(0, bufa)

        def chunk2(t, carry):
            step(2 * t, bufa, bufb)
            step(2 * t + 1, bufb, bufa)
            return carry

        lax.fori_loop(0, NCH // 2, chunk2, 0)
        plsc.subcore_barrier()
        pltpu.sync_copy(acc.at[pl.ds(sid * NPT, NPT)],
                        out_hbm.at[cid, pl.ds(sid * NPT, NPT)])

    return ek


def kernel(y, y_edge_index, Wl1, Wr1, att1, b1, Wl2, Wr2, att2, b2):
    n = y.shape[0]
    e = y_edge_index.shape[1]
    ei = y_edge_index.astype(jnp.int32)

    expand = jnp.zeros((8, 128), jnp.float32)
    hidx = jnp.arange(128) // 32
    expand = expand.at[hidx, jnp.arange(128)].set(1.0)

    b1m = jnp.broadcast_to(b1.reshape(1, -1), (8, 128))
    b2m = jnp.broadcast_to(b2.reshape(1, -1), (8, 64))

    xl1, xr1 = _mm2(y, Wl1, Wr1)
    acc1 = _make_edge_kernel(n, e, 128, 4)(xl1, xr1, ei, att1.reshape(-1))
    xl2, xr2 = _mid(acc1[0], acc1[1], expand, b1m, Wl2, Wr2)

    acc2 = _make_edge_kernel(n, e, 64, 1)(xl2, xr2, ei, att2.reshape(-1))
    return _fin(acc2[0], acc2[1], b2m)


# speedup vs baseline: 1.0983x; 1.0722x over previous
import functools

import jax
import jax.numpy as jnp
from jax import lax
from jax.experimental import pallas as pl
from jax.experimental.pallas import tpu as pltpu
from jax.experimental.pallas import tpu_sc as plsc


_ROWS = 400


def _mm2_body(y_ref, wl_ref, wr_ref, xl_ref, xr_ref):
    yb = y_ref[...]
    xl_ref[...] = jnp.dot(yb, wl_ref[...], preferred_element_type=jnp.float32)
    xr_ref[...] = jnp.dot(yb, wr_ref[...], preferred_element_type=jnp.float32)


def _mm2(y, Wl, Wr):
    n, p = y.shape
    d = Wl.shape[1]
    grid = n // _ROWS
    return pl.pallas_call(
        _mm2_body,
        grid=(grid,),
        in_specs=[
            pl.BlockSpec((_ROWS, p), lambda i: (i, 0)),
            pl.BlockSpec((p, d), lambda i: (0, 0)),
            pl.BlockSpec((p, d), lambda i: (0, 0)),
        ],
        out_specs=[
            pl.BlockSpec((_ROWS, d), lambda i: (i, 0)),
            pl.BlockSpec((_ROWS, d), lambda i: (i, 0)),
        ],
        out_shape=[
            jax.ShapeDtypeStruct((n, d), jnp.float32),
            jax.ShapeDtypeStruct((n, d), jnp.float32),
        ],
    )(y, Wl, Wr)


def _mid_body(a0_ref, a1_ref, exp_ref, b_ref, wl_ref, wr_ref, xl_ref, xr_ref):
    s = a0_ref[...] + a1_ref[...]
    msg = s[:, :128]
    r8 = 1.0 / (s[:, 128:136] + 1e-16)
    rw = jnp.dot(r8, exp_ref[...], preferred_element_type=jnp.float32)
    out1 = msg * rw + b_ref[...][0:1, :]
    h = jnp.where(out1 > 0.0, out1, jnp.exp(out1) - 1.0)
    xl_ref[...] = jnp.dot(h, wl_ref[...], preferred_element_type=jnp.float32)
    xr_ref[...] = jnp.dot(h, wr_ref[...], preferred_element_type=jnp.float32)


def _mid(a0, a1, expand, b1, Wl2, Wr2):
    n, w = a0.shape
    z = Wl2.shape[1]
    grid = n // _ROWS
    return pl.pallas_call(
        _mid_body,
        grid=(grid,),
        in_specs=[
            pl.BlockSpec((_ROWS, w), lambda i: (i, 0)),
            pl.BlockSpec((_ROWS, w), lambda i: (i, 0)),
            pl.BlockSpec((8, 128), lambda i: (0, 0)),
            pl.BlockSpec((8, 128), lambda i: (0, 0)),
            pl.BlockSpec((128, z), lambda i: (0, 0)),
            pl.BlockSpec((128, z), lambda i: (0, 0)),
        ],
        out_specs=[
            pl.BlockSpec((_ROWS, z), lambda i: (i, 0)),
            pl.BlockSpec((_ROWS, z), lambda i: (i, 0)),
        ],
        out_shape=[
            jax.ShapeDtypeStruct((n, z), jnp.float32),
            jax.ShapeDtypeStruct((n, z), jnp.float32),
        ],
    )(a0, a1, expand, b1, Wl2, Wr2)


def _fin_body(a0_ref, a1_ref, b_ref, z_ref):
    s = a0_ref[...] + a1_ref[...]
    msg = s[:, :64]
    r = 1.0 / (s[:, 64:65] + 1e-16)
    z_ref[...] = msg * r + b_ref[...][0:1, :]


def _fin(a0, a1, b2):
    n, w = a0.shape
    z = b2.shape[1]
    grid = n // _ROWS
    return pl.pallas_call(
        _fin_body,
        grid=(grid,),
        in_specs=[
            pl.BlockSpec((_ROWS, w), lambda i: (i, 0)),
            pl.BlockSpec((_ROWS, w), lambda i: (i, 0)),
            pl.BlockSpec((8, z), lambda i: (0, 0)),
        ],
        out_specs=pl.BlockSpec((_ROWS, z), lambda i: (i, 0)),
        out_shape=jax.ShapeDtypeStruct((n, z), jnp.float32),
    )(a0, a1, b2)


_NC, _NS = 2, 16
_K = 40


def _make_edge_kernel(N, E, D, H):
    ACCW = D + 16
    NW = _NC * _NS
    EPT = E // NW
    NCH = EPT // _K
    NPT = N // _NS
    NJ = D // 16
    JH = NJ // H
    assert EPT % _K == 0 and NCH % 2 == 0 and NJ % H == 0

    mesh = plsc.VectorSubcoreMesh(core_axis_name="c", subcore_axis_name="s")

    @functools.partial(
        pl.kernel,
        out_type=jax.ShapeDtypeStruct((_NC, N, ACCW), jnp.float32),
        mesh=mesh,
        compiler_params=pltpu.CompilerParams(
            use_tc_tiling_on_sc=False, needs_layout_passes=False),
        scratch_types=[
            pltpu.VMEM_SHARED((N, ACCW), jnp.float32),
            pltpu.VMEM((_K,), jnp.int32),
            pltpu.VMEM((_K,), jnp.int32),
            pltpu.VMEM((_K,), jnp.int32),
            pltpu.VMEM((_K,), jnp.int32),
            pltpu.VMEM((_K, D), jnp.float32),
            pltpu.VMEM((_K, D), jnp.float32),
            pltpu.VMEM((_K, D), jnp.float32),
            pltpu.VMEM((_K, D), jnp.float32),
            pltpu.VMEM((_K, ACCW), jnp.float32),
            pltpu.VMEM((D,), jnp.float32),
            pltpu.SemaphoreType.DMA,
            pltpu.SemaphoreType.DMA,
            pltpu.SemaphoreType.DMA,
            pltpu.SemaphoreType.DMA,
        ],
    )
    def ek(xl_hbm, xr_hbm, ei_hbm, att_hbm, out_hbm,
           acc, src_a, dst_a, src_b, dst_b, xlb_a, xrb_a, xlb_b, xrb_b,
           msgb, attb, sxl_a, sxr_a, sxl_b, sxr_b):
        cid = lax.axis_index("c")
        sid = lax.axis_index("s")
        wid = cid * _NS + sid
        zv = jnp.zeros((16,), jnp.float32)
        lane = lax.iota(jnp.int32, 16)

        nseg = ACCW // 16

        def zbody(t, carry):
            msgb[t // nseg, pl.ds((t % nseg) * 16, 16)] = zv
            return carry

        lax.fori_loop(0, _K * nseg, zbody, 0)
        for j in range(NPT // _K):
            pltpu.sync_copy(msgb, acc.at[pl.ds(sid * NPT + j * _K, _K)])
        zrem = NPT % _K
        if zrem:
            pltpu.sync_copy(
                msgb.at[pl.ds(0, zrem)],
                acc.at[pl.ds(sid * NPT + (NPT // _K) * _K, zrem)])
        pltpu.sync_copy(att_hbm, attb)
        plsc.subcore_barrier()

        attv = [attb[pl.ds(16 * j, 16)] for j in range(NJ)]

        bufa = (src_a, dst_a, xlb_a, xrb_a, sxl_a, sxr_a)
        bufb = (src_b, dst_b, xlb_b, xrb_b, sxl_b, sxr_b)

        def issue(i, buf):
            src_v, dst_v, xlb, xrb, sxl, sxr = buf
            ebase = wid * EPT + i * _K
            pltpu.sync_copy(ei_hbm.at[0, pl.ds(ebase, _K)], src_v)
            pltpu.sync_copy(ei_hbm.at[1, pl.ds(ebase, _K)], dst_v)
            pltpu.async_copy(xl_hbm.at[src_v], xlb, sxl)
            pltpu.async_copy(xr_hbm.at[dst_v], xrb, sxr)

        def step(i, cur, nxt):
            src_v, dst_v, xlb, xrb, sxl, sxr = cur

            @pl.when(i + 1 < NCH)
            def _():
                issue(i + 1, nxt)

            pltpu.make_async_copy(xl_hbm.at[src_v], xlb, sxl).wait()
            pltpu.make_async_copy(xr_hbm.at[dst_v], xrb, sxr).wait()

            def edge(k):
                xs = [xlb[k, pl.ds(16 * j, 16)] for j in range(NJ)]
                ts = []
                for j in range(NJ):
                    s = xs[j] + xrb[k, pl.ds(16 * j, 16)]
                    m = jnp.maximum(s, 0.2 * s)
                    ts.append(m * attv[j])
                exb = []
                for h in range(H):
                    u = ts[h * JH]
                    for q in range(1, JH):
                        u = u + ts[h * JH + q]
                    eh = jnp.sum(u)
                    exb.append(jnp.exp(jnp.full((16,), eh, jnp.float32)))
                for j in range(NJ):
                    msgb[k, pl.ds(16 * j, 16)] = xs[j] * exb[j // JH]
                evec = zv
                for h in range(H):
                    evec = jnp.where(lane == h, exb[h], evec)
                msgb[k, pl.ds(D, 16)] = evec

            plsc.parallel_loop(0, _K, 1, unroll=4)(lambda k: edge(k))
            pltpu.sync_copy(msgb, acc.at[dst_v], add=True)

        issue(0, bufa)

        def chunk2(t, carry):
            step(2 * t, bufa, bufb)
            step(2 * t + 1, bufb, bufa)
            return carry

        lax.fori_loop(0, NCH // 2, chunk2, 0)
        plsc.subcore_barrier()
        pltpu.sync_copy(acc.at[pl.ds(sid * NPT, NPT)],
                        out_hbm.at[cid, pl.ds(sid * NPT, NPT)])

    return ek


def kernel(y, y_edge_index, Wl1, Wr1, att1, b1, Wl2, Wr2, att2, b2):
    n = y.shape[0]
    e = y_edge_index.shape[1]
    ei = y_edge_index.astype(jnp.int32)

    expand = jnp.zeros((8, 128), jnp.float32)
    hidx = jnp.arange(128) // 32
    expand = expand.at[hidx, jnp.arange(128)].set(1.0)

    b1m = jnp.broadcast_to(b1.reshape(1, -1), (8, 128))
    b2m = jnp.broadcast_to(b2.reshape(1, -1), (8, 64))

    xl1, xr1 = _mm2(y, Wl1, Wr1)
    acc1 = _make_edge_kernel(n, e, 128, 4)(xl1, xr1, ei, att1.reshape(-1))
    xl2, xr2 = _mid(acc1[0], acc1[1], expand, b1m, Wl2, Wr2)

    acc2 = _make_edge_kernel(n, e, 64, 1)(xl2, xr2, ei, att2.reshape(-1))
    return _fin(acc2[0], acc2[1], b2m)
